# baseline (device time: 13430 ns/iter reference)
import jax
import jax.numpy as jnp
from jax import lax
from jax.experimental import pallas as pl
from jax.experimental.pallas import tpu as pltpu

N_DEV = 4
K = 32
W = 8
T = 8


def kernel(x):
    m, n = x.shape
    P = m // K
    out_dtype = jnp.bfloat16

    def body(x_ref, o_ref, in_vmem, out_vmem, halo_ref, in_sems, out_sems,
             send_sems, recv_sems):
        my = lax.axis_index("i")
        has_left = my > 0
        has_right = my < N_DEV - 1

        def in_copy(c):
            return pltpu.make_async_copy(
                x_ref.at[pl.ds(c * P, P), :],
                in_vmem.at[pl.ds(c * P, P), :],
                in_sems.at[c])

        def out_copy(c):
            return pltpu.make_async_copy(
                out_vmem.at[pl.ds(c * P, P), :],
                o_ref.at[pl.ds(c * P, P), :],
                out_sems.at[c])

        def fix_row(r):
            out_vmem[pl.ds(r, 1), :] = (
                0.25 * in_vmem[pl.ds(r - 1, 1), :]
                + 0.5 * in_vmem[pl.ds(r, 1), :]
                + 0.25 * in_vmem[pl.ds(r + 1, 1), :]
            ).astype(out_dtype)

        def stencil(lo, rows, patch_first, patch_last):
            v = in_vmem[pl.ds(lo, rows), :]
            out_vmem[pl.ds(lo, rows), :] = (
                0.25 * pltpu.roll(v, 1, 0)
                + 0.5 * v
                + 0.25 * pltpu.roll(v, rows - 1, 0)
            ).astype(out_dtype)
            if patch_first:
                fix_row(lo)
            if patch_last:
                fix_row(lo + rows - 1)

        for c in range(W):
            in_copy(c).start()

        barrier = pltpu.get_barrier_semaphore()

        @pl.when(has_left)
        def _():
            pl.semaphore_signal(
                barrier, inc=1,
                device_id=(my - 1,), device_id_type=pl.DeviceIdType.MESH,
            )

        @pl.when(has_right)
        def _():
            pl.semaphore_signal(
                barrier, inc=1,
                device_id=(my + 1,), device_id_type=pl.DeviceIdType.MESH,
            )

        @pl.when(has_left)
        def _():
            pl.semaphore_wait(barrier, 1)

        @pl.when(has_right)
        def _():
            pl.semaphore_wait(barrier, 1)

        send_left = pltpu.make_async_remote_copy(
            src_ref=x_ref.at[pl.ds(0, T), :],
            dst_ref=halo_ref.at[1],
            send_sem=send_sems.at[0],
            recv_sem=recv_sems.at[1],
            device_id=(my - 1,),
            device_id_type=pl.DeviceIdType.MESH,
        )
        send_right = pltpu.make_async_remote_copy(
            src_ref=x_ref.at[pl.ds(m - T, T), :],
            dst_ref=halo_ref.at[0],
            send_sem=send_sems.at[1],
            recv_sem=recv_sems.at[0],
            device_id=(my + 1,),
            device_id_type=pl.DeviceIdType.MESH,
        )

        @pl.when(has_left)
        def _():
            send_left.start()

        @pl.when(has_right)
        def _():
            send_right.start()

        in_copy(0).wait()
        for c in range(1, K):
            if c + W - 1 < K:
                in_copy(c + W - 1).start()
            in_copy(c).wait()
            if c == 1:
                stencil(0, P, patch_first=False, patch_last=True)
            else:
                stencil((c - 1) * P, P, patch_first=True, patch_last=True)
                out_copy(c - 1).start()

        stencil((K - 1) * P, P, patch_first=True, patch_last=False)

        @pl.when(jnp.logical_not(has_right))
        def _():
            out_vmem[pl.ds(m - 1, 1), :] = (
                in_vmem[pl.ds(m - 1, 1), :].astype(out_dtype))

        @pl.when(has_right)
        def _():
            send_left.wait_recv()
            out_vmem[pl.ds(m - 1, 1), :] = (
                0.25 * in_vmem[pl.ds(m - 2, 1), :]
                + 0.5 * in_vmem[pl.ds(m - 1, 1), :]
                + 0.25 * halo_ref[1, pl.ds(0, 1), :]
            ).astype(out_dtype)

        out_copy(K - 1).start()

        @pl.when(jnp.logical_not(has_left))
        def _():
            out_vmem[pl.ds(0, 1), :] = (
                in_vmem[pl.ds(0, 1), :].astype(out_dtype))

        @pl.when(has_left)
        def _():
            send_right.wait_recv()
            out_vmem[pl.ds(0, 1), :] = (
                0.25 * halo_ref[0, pl.ds(T - 1, 1), :]
                + 0.5 * in_vmem[pl.ds(0, 1), :]
                + 0.25 * in_vmem[pl.ds(1, 1), :]
            ).astype(out_dtype)

        out_copy(0).start()

        for c in range(K):
            out_copy(c).wait()

        @pl.when(has_left)
        def _():
            send_left.wait_send()

        @pl.when(has_right)
        def _():
            send_right.wait_send()

    return pl.pallas_call(
        body,
        out_shape=jax.ShapeDtypeStruct((m, n), out_dtype),
        in_specs=[pl.BlockSpec(memory_space=pl.ANY)],
        out_specs=pl.BlockSpec(memory_space=pl.ANY),
        scratch_shapes=[
            pltpu.VMEM((m, n), x.dtype),
            pltpu.VMEM((m, n), out_dtype),
            pltpu.VMEM((2, T, n), x.dtype),
            pltpu.SemaphoreType.DMA((K,)),
            pltpu.SemaphoreType.DMA((K,)),
            pltpu.SemaphoreType.DMA((2,)),
            pltpu.SemaphoreType.DMA((2,)),
        ],
        compiler_params=pltpu.CompilerParams(collective_id=0),
    )(x)


# device time: 13257 ns/iter; 1.0130x vs baseline; 1.0130x over previous
import jax
import jax.numpy as jnp
from jax import lax
from jax.experimental import pallas as pl
from jax.experimental.pallas import tpu as pltpu

N_DEV = 4
K = 16
W = 4
T = 8


def kernel(x):
    m, n = x.shape
    P = m // K
    out_dtype = jnp.bfloat16

    def body(x_ref, o_ref, in_vmem, out_vmem, halo_ref, in_sems, out_sems,
             send_sems, recv_sems):
        my = lax.axis_index("i")
        has_left = my > 0
        has_right = my < N_DEV - 1

        def in_copy(c):
            return pltpu.make_async_copy(
                x_ref.at[pl.ds(c * P, P), :],
                in_vmem.at[pl.ds(c * P, P), :],
                in_sems.at[c])

        def out_copy(c):
            return pltpu.make_async_copy(
                out_vmem.at[pl.ds(c * P, P), :],
                o_ref.at[pl.ds(c * P, P), :],
                out_sems.at[c])

        def fix_row(r):
            out_vmem[pl.ds(r, 1), :] = (
                0.25 * in_vmem[pl.ds(r - 1, 1), :]
                + 0.5 * in_vmem[pl.ds(r, 1), :]
                + 0.25 * in_vmem[pl.ds(r + 1, 1), :]
            ).astype(out_dtype)

        def stencil(lo, rows, patch_first, patch_last):
            v = in_vmem[pl.ds(lo, rows), :]
            out_vmem[pl.ds(lo, rows), :] = (
                0.25 * pltpu.roll(v, 1, 0)
                + 0.5 * v
                + 0.25 * pltpu.roll(v, rows - 1, 0)
            ).astype(out_dtype)
            if patch_first:
                fix_row(lo)
            if patch_last:
                fix_row(lo + rows - 1)

        for c in range(W):
            in_copy(c).start()

        barrier = pltpu.get_barrier_semaphore()

        @pl.when(has_left)
        def _():
            pl.semaphore_signal(
                barrier, inc=1,
                device_id=(my - 1,), device_id_type=pl.DeviceIdType.MESH,
            )

        @pl.when(has_right)
        def _():
            pl.semaphore_signal(
                barrier, inc=1,
                device_id=(my + 1,), device_id_type=pl.DeviceIdType.MESH,
            )

        @pl.when(has_left)
        def _():
            pl.semaphore_wait(barrier, 1)

        @pl.when(has_right)
        def _():
            pl.semaphore_wait(barrier, 1)

        send_left = pltpu.make_async_remote_copy(
            src_ref=x_ref.at[pl.ds(0, T), :],
            dst_ref=halo_ref.at[1],
            send_sem=send_sems.at[0],
            recv_sem=recv_sems.at[1],
            device_id=(my - 1,),
            device_id_type=pl.DeviceIdType.MESH,
        )
        send_right = pltpu.make_async_remote_copy(
            src_ref=x_ref.at[pl.ds(m - T, T), :],
            dst_ref=halo_ref.at[0],
            send_sem=send_sems.at[1],
            recv_sem=recv_sems.at[0],
            device_id=(my + 1,),
            device_id_type=pl.DeviceIdType.MESH,
        )

        @pl.when(has_left)
        def _():
            send_left.start()

        @pl.when(has_right)
        def _():
            send_right.start()

        in_copy(0).wait()
        for c in range(1, K):
            if c + W - 1 < K:
                in_copy(c + W - 1).start()
            in_copy(c).wait()
            if c == 1:
                stencil(0, P, patch_first=False, patch_last=True)
            else:
                stencil((c - 1) * P, P, patch_first=True, patch_last=True)
                out_copy(c - 1).start()

        stencil((K - 1) * P, P, patch_first=True, patch_last=False)

        @pl.when(jnp.logical_not(has_right))
        def _():
            out_vmem[pl.ds(m - 1, 1), :] = (
                in_vmem[pl.ds(m - 1, 1), :].astype(out_dtype))

        @pl.when(has_right)
        def _():
            send_left.wait_recv()
            out_vmem[pl.ds(m - 1, 1), :] = (
                0.25 * in_vmem[pl.ds(m - 2, 1), :]
                + 0.5 * in_vmem[pl.ds(m - 1, 1), :]
                + 0.25 * halo_ref[1, pl.ds(0, 1), :]
            ).astype(out_dtype)

        out_copy(K - 1).start()

        @pl.when(jnp.logical_not(has_left))
        def _():
            out_vmem[pl.ds(0, 1), :] = (
                in_vmem[pl.ds(0, 1), :].astype(out_dtype))

        @pl.when(has_left)
        def _():
            send_right.wait_recv()
            out_vmem[pl.ds(0, 1), :] = (
                0.25 * halo_ref[0, pl.ds(T - 1, 1), :]
                + 0.5 * in_vmem[pl.ds(0, 1), :]
                + 0.25 * in_vmem[pl.ds(1, 1), :]
            ).astype(out_dtype)

        out_copy(0).start()

        for c in range(K):
            out_copy(c).wait()

        @pl.when(has_left)
        def _():
            send_left.wait_send()

        @pl.when(has_right)
        def _():
            send_right.wait_send()

    return pl.pallas_call(
        body,
        out_shape=jax.ShapeDtypeStruct((m, n), out_dtype),
        in_specs=[pl.BlockSpec(memory_space=pl.ANY)],
        out_specs=pl.BlockSpec(memory_space=pl.ANY),
        scratch_shapes=[
            pltpu.VMEM((m, n), x.dtype),
            pltpu.VMEM((m, n), out_dtype),
            pltpu.VMEM((2, T, n), x.dtype),
            pltpu.SemaphoreType.DMA((K,)),
            pltpu.SemaphoreType.DMA((K,)),
            pltpu.SemaphoreType.DMA((2,)),
            pltpu.SemaphoreType.DMA((2,)),
        ],
        compiler_params=pltpu.CompilerParams(collective_id=0),
    )(x)
